# baseline (speedup 1.0000x reference)
import jax
import jax.numpy as jnp
from jax.experimental import pallas as pl

N = 10000
D = 128
H = 128
C = 2
BM = 400


def _bn_kernel(f_ref, g_ref, b_ref, o_ref):
    x = f_ref[...]
    mu = jnp.mean(x, axis=0, keepdims=True)
    var = jnp.mean((x - mu) * (x - mu), axis=0, keepdims=True)
    o_ref[...] = (x - mu) * jax.lax.rsqrt(var + 1e-5) * g_ref[...] + b_ref[...]


def _mm_kernel(a_ref, x_ref, o_ref):
    o_ref[...] = jnp.dot(a_ref[...], x_ref[...],
                         preferred_element_type=jnp.float32)


def _final_kernel(a_ref, x_ref, w1_ref, b1_ref, a1_ref, w2_ref, b2_ref,
                  a2_ref, w3_ref, b3_ref, o_ref):
    x2 = jnp.dot(a_ref[...], x_ref[...], preferred_element_type=jnp.float32)
    h = jnp.dot(x2, w1_ref[...], preferred_element_type=jnp.float32) + b1_ref[...]
    h = jnp.where(h >= 0, h, a1_ref[0, 0] * h)
    h = jnp.dot(h, w2_ref[...], preferred_element_type=jnp.float32) + b2_ref[...]
    h = jnp.where(h >= 0, h, a2_ref[0, 0] * h)
    h = jnp.dot(h, w3_ref[...], preferred_element_type=jnp.float32) + b3_ref[...]
    m = jnp.max(h, axis=1, keepdims=True)
    s = h - m
    lse = jnp.log(jnp.sum(jnp.exp(s), axis=1, keepdims=True))
    o_ref[...] = s - lse


def kernel(feature, adj, gamma, beta, W1, b1, a1, W2, b2, a2, W3, b3):
    gamma2 = gamma.reshape(1, D)
    beta2 = beta.reshape(1, D)
    b1_2 = b1.reshape(1, H)
    b2_2 = b2.reshape(1, H)
    b3_2 = b3.reshape(1, C)
    a1_2 = a1.reshape(1, 1)
    a2_2 = a2.reshape(1, 1)

    xn = pl.pallas_call(
        _bn_kernel,
        out_shape=jax.ShapeDtypeStruct((N, D), jnp.float32),
    )(feature, gamma2, beta2)

    nb = N // BM
    x1 = pl.pallas_call(
        _mm_kernel,
        grid=(nb,),
        in_specs=[
            pl.BlockSpec((BM, N), lambda i: (i, 0)),
            pl.BlockSpec((N, D), lambda i: (0, 0)),
        ],
        out_specs=pl.BlockSpec((BM, D), lambda i: (i, 0)),
        out_shape=jax.ShapeDtypeStruct((N, D), jnp.float32),
    )(adj, xn)

    out = pl.pallas_call(
        _final_kernel,
        grid=(nb,),
        in_specs=[
            pl.BlockSpec((BM, N), lambda i: (i, 0)),
            pl.BlockSpec((N, D), lambda i: (0, 0)),
            pl.BlockSpec((D, H), lambda i: (0, 0)),
            pl.BlockSpec((1, H), lambda i: (0, 0)),
            pl.BlockSpec((1, 1), lambda i: (0, 0)),
            pl.BlockSpec((H, H), lambda i: (0, 0)),
            pl.BlockSpec((1, H), lambda i: (0, 0)),
            pl.BlockSpec((1, 1), lambda i: (0, 0)),
            pl.BlockSpec((H, C), lambda i: (0, 0)),
            pl.BlockSpec((1, C), lambda i: (0, 0)),
        ],
        out_specs=pl.BlockSpec((BM, C), lambda i: (i, 0)),
        out_shape=jax.ShapeDtypeStruct((N, C), jnp.float32),
    )(adj, x1, W1, b1_2, a1_2, W2, b2_2, a2_2, W3, b3_2)
    return out
